# baseline (speedup 1.0000x reference)
import jax
import jax.numpy as jnp
from jax import lax
from jax.experimental import pallas as pl
from jax.experimental.pallas import tpu as pltpu
from jax.experimental.pallas import tpu_sc as plsc
import functools

_B, _N = 16, 256
_IMG_D, _PC_D = 2048, 768
_VIS_D, _SP_D = 768, 8
_THR = 0.5
_EPS = 1e-8
_NEG = -1e30
_F32 = jnp.float32
_BPG = 2
_BPG2 = 4
_BF16 = jnp.bfloat16


def _sim_body(img_ref, pc_ref, pimg_ref, ppc_ref, mask_ref, s_ref):
    img = img_ref[0]
    pc = pc_ref[0]
    pimg = pimg_ref[0]
    ppc = ppc_ref[0]
    dn = (((1,), (1,)), ((), ()))
    dot = lax.dot_general(img, pimg, dn, preferred_element_type=_F32)
    dot += lax.dot_general(pc, ppc, dn, preferred_element_type=_F32)
    sq = lambda a: jnp.sum(a.astype(_F32) ** 2, 1, keepdims=True)
    nf = jnp.maximum(jnp.sqrt(sq(img) + sq(pc)), _EPS)
    npr = jnp.maximum(jnp.sqrt(sq(pimg) + sq(ppc)), _EPS)
    sim = dot / (nf * npr.reshape(1, _N))
    s_ref[0] = jnp.where(mask_ref[0] != 0, sim, _NEG)


def _vperm16(x, perm):
    dnums = lax.GatherDimensionNumbers(offset_dims=(),
                                       collapsed_slice_dims=(0,),
                                       start_index_map=(0,))
    return lax.gather(x, perm.reshape(16, 1), dnums, slice_sizes=(1,),
                      mode=lax.GatherScatterMode.PROMISE_IN_BOUNDS)


def _bcast_max16(x):
    iota = lax.broadcasted_iota(jnp.int32, (16,), 0)
    for sh in (1, 2, 4, 8):
        x = jnp.maximum(x, _vperm16(x, jnp.bitwise_xor(iota, sh)))
    return x


def _bcast_min16i(x):
    iota = lax.broadcasted_iota(jnp.int32, (16,), 0)
    for sh in (1, 2, 4, 8):
        x = jnp.minimum(x, _vperm16(x, jnp.bitwise_xor(iota, sh)))
    return x


def _sc_match_body(s_hbm, p_hbm, flag_hbm, s_v, p_blk, vis_v, mx_v,
                   allmx_v, flag_v, mx_sh, sem):
    cid = lax.axis_index("c")
    sid = lax.axis_index("s")
    active = cid == 0
    iota = lax.broadcasted_iota(jnp.int32, (16,), 0)

    @pl.when(active)
    def _local_max():
        pltpu.sync_copy(s_hbm.at[sid], s_v)

        def rowmax(i, acc):
            for k in range(16):
                acc = jnp.maximum(acc, s_v[i, pl.ds(k * 16, 16)])
            return acc

        mx_v[...] = lax.fori_loop(0, _N, rowmax,
                                  jnp.full((16,), _NEG, _F32))
        pltpu.sync_copy(mx_v, mx_sh.at[sid])

    plsc.subcore_barrier()

    @pl.when(active)
    def _decide():
        pltpu.sync_copy(mx_sh, allmx_v)
        g = jnp.full((16,), _NEG, _F32)
        for r in range(16):
            g = jnp.maximum(g, allmx_v[r])
        any_hit = _bcast_max16(g)[0] >= _THR

        @pl.when(sid == 0)
        def _flag():
            flag_v[0] = jnp.ones((16,), jnp.int32) * any_hit.astype(jnp.int32)
            pltpu.sync_copy(flag_v, flag_hbm)

        @pl.when(any_hit)
        def _slow():
            for k in range(16):
                vis_v[pl.ds(k * 16, 16)] = jnp.zeros((16,), _F32)

            def blk(gblk, c0):
                def zrow(r, c1):
                    for k in range(16):
                        p_blk[r, pl.ds(k * 16, 16)] = jnp.zeros((16,), _F32)
                    return c1

                lax.fori_loop(0, 64, zrow, 0)

                def step(ii, c2):
                    i = gblk * 64 + ii
                    vmax = jnp.full((16,), _NEG, _F32)
                    for k in range(16):
                        vmax = jnp.maximum(
                            vmax, s_v[i, pl.ds(k * 16, 16)]
                            + vis_v[pl.ds(k * 16, 16)])
                    mv = _bcast_max16(vmax)
                    m = mv[0]
                    cmin = jnp.full((16,), 1024, jnp.int32)
                    for k in range(16):
                        v = (s_v[i, pl.ds(k * 16, 16)]
                             + vis_v[pl.ds(k * 16, 16)])
                        cmin = jnp.minimum(
                            cmin, jnp.where(v == mv, iota + k * 16, 1024))
                    jv = _bcast_min16i(cmin)

                    @pl.when(m >= _THR)
                    def _mark():
                        for k in range(16):
                            sel = (iota + k * 16) == jv
                            vis_v[pl.ds(k * 16, 16)] = jnp.where(
                                sel, _NEG, vis_v[pl.ds(k * 16, 16)])
                            p_blk[ii, pl.ds(k * 16, 16)] = jnp.where(
                                sel, 1.0, 0.0).astype(_F32)
                    return c2

                lax.fori_loop(0, 64, step, 0)
                pltpu.sync_copy(p_blk, p_hbm.at[sid, pl.ds(gblk * 64, 64)])
                return c0

            lax.fori_loop(0, 4, blk, 0)


def _sc_match(s_t):
    mesh = plsc.VectorSubcoreMesh(core_axis_name="c", subcore_axis_name="s")
    kern = functools.partial(
        pl.kernel, mesh=mesh,
        out_type=[jax.ShapeDtypeStruct((_B, _N, _N), _F32),
                  jax.ShapeDtypeStruct((1, 16), jnp.int32)],
        scratch_types=[
            pltpu.VMEM((_N, _N), _F32),
            pltpu.VMEM((64, _N), _F32),
            pltpu.VMEM((_N,), _F32),
            pltpu.VMEM((16,), _F32),
            pltpu.VMEM((16, 16), _F32),
            pltpu.VMEM((1, 16), jnp.int32),
            pltpu.VMEM_SHARED((16, 16), _F32),
            pltpu.SemaphoreType.DMA,
        ],
    )(_sc_match_body)
    return kern(s_t)


def _layer_norm(x, g, b):
    m = jnp.mean(x, axis=1, keepdims=True)
    v = jnp.mean((x - m) ** 2, axis=1, keepdims=True)
    return (x - m) / jnp.sqrt(v + 1e-5) * g + b


def _img_chain_body(flag_ref, p_ref, img_ref, pimg_ref, w1a_hbm, w1b_hbm,
                    w2_hbm, w3_hbm, b1_ref, b2_ref, b3_ref, g_ref, bb_ref,
                    hi_ref, w1a_ref, w1b_ref, w2_ref, w3_ref, sem):
    @pl.when(pl.program_id(0) == 0)
    def _load_weights():
        cps = [pltpu.make_async_copy(w1a_hbm, w1a_ref, sem),
               pltpu.make_async_copy(w1b_hbm, w1b_ref, sem),
               pltpu.make_async_copy(w2_hbm, w2_ref, sem),
               pltpu.make_async_copy(w3_hbm, w3_ref, sem)]
        for c in cps:
            c.start()
        for c in cps:
            c.wait()

    x = img_ref[...].reshape(_BPG * _N, _IMG_D)
    base = jnp.dot(x, w1a_ref[...], preferred_element_type=_F32) + b1_ref[...]

    def _with_prev():
        oimg = jnp.concatenate(
            [jnp.dot(p_ref[k].astype(_BF16), pimg_ref[k],
                     preferred_element_type=_F32) for k in range(_BPG)], axis=0)
        return base + jnp.dot(oimg.astype(_BF16), w1b_ref[...],
                              preferred_element_type=_F32)

    h = lax.cond(flag_ref[0, 0] == 1, _with_prev, lambda: base)
    h = jnp.maximum(h, 0.0).astype(_BF16)
    h = jnp.dot(h, w2_ref[...], preferred_element_type=_F32) + b2_ref[...]
    h = jnp.dot(h.astype(_BF16), w3_ref[...], preferred_element_type=_F32)
    h = h + b3_ref[...]
    hi_ref[...] = _layer_norm(h, g_ref[...], bb_ref[...]).reshape(
        _BPG, _N, _VIS_D)


def _pcfu_body(flag_ref, p_ref, pc_ref, ppc_ref, psp_ref, hi_ref,
               pw1a_hbm, pw1b_hbm, pw2_hbm, pw3_hbm,
               fw1a_hbm, fw1b_hbm, fw2_hbm,
               pb1_ref, pb2_ref, pb3_ref, plg_ref, plb_ref,
               fb1_ref, fb2_ref, flg_ref, flb_ref, vis_ref, sp_ref,
               pw1a_ref, pw1b_ref, pw2_ref, pw3_ref,
               fw1a_ref, fw1b_ref, fw2_ref, sem):
    @pl.when(pl.program_id(0) == 0)
    def _load_weights():
        cps = [pltpu.make_async_copy(pw1a_hbm, pw1a_ref, sem),
               pltpu.make_async_copy(pw1b_hbm, pw1b_ref, sem),
               pltpu.make_async_copy(pw2_hbm, pw2_ref, sem),
               pltpu.make_async_copy(pw3_hbm, pw3_ref, sem),
               pltpu.make_async_copy(fw1a_hbm, fw1a_ref, sem),
               pltpu.make_async_copy(fw1b_hbm, fw1b_ref, sem),
               pltpu.make_async_copy(fw2_hbm, fw2_ref, sem)]
        for c in cps:
            c.start()
        for c in cps:
            c.wait()

    matched = flag_ref[0, 0] == 1

    @pl.when(matched)
    def _sp_gather():
        for k in range(_BPG2):
            sp_ref[k] = jnp.dot(p_ref[k], psp_ref[k],
                                preferred_element_type=_F32)

    @pl.when(jnp.logical_not(matched))
    def _sp_zero():
        sp_ref[...] = jnp.zeros((_BPG2, _N, _SP_D), _F32)

    xpc = pc_ref[...].reshape(_BPG2 * _N, _PC_D)
    pbase = jnp.dot(xpc, pw1a_ref[...], preferred_element_type=_F32) + pb1_ref[...]

    def _with_prev():
        opc = jnp.concatenate(
            [jnp.dot(p_ref[k].astype(_BF16), ppc_ref[k],
                     preferred_element_type=_F32) for k in range(_BPG2)], axis=0)
        return pbase + jnp.dot(opc.astype(_BF16), pw1b_ref[...],
                               preferred_element_type=_F32)

    h = lax.cond(matched, _with_prev, lambda: pbase)
    h = jnp.maximum(h, 0.0).astype(_BF16)
    h = jnp.dot(h, pw2_ref[...], preferred_element_type=_F32) + pb2_ref[...]
    h = jnp.dot(h.astype(_BF16), pw3_ref[...], preferred_element_type=_F32)
    h = h + pb3_ref[...]
    hp = _layer_norm(h, plg_ref[...], plb_ref[...])

    h = (jnp.dot(hi_ref[...].reshape(_BPG2 * _N, _VIS_D).astype(_BF16),
                 fw1a_ref[...], preferred_element_type=_F32)
         + jnp.dot(hp.astype(_BF16), fw1b_ref[...], preferred_element_type=_F32)
         + fb1_ref[...])
    h = jnp.maximum(h, 0.0).astype(_BF16)
    h = jnp.dot(h, fw2_ref[...], preferred_element_type=_F32) + fb2_ref[...]
    vis_ref[...] = _layer_norm(h, flg_ref[...], flb_ref[...]).reshape(
        _BPG2, _N, _VIS_D)


def kernel(image_feature, point_cloud_feature, prev_image_feature,
           prev_point_cloud_feature, rel_dist_mask, prev_spatial,
           img_w1, img_b1, img_w2, img_b2, img_w3, img_b3, img_ln_g, img_ln_b,
           pc_w1, pc_b1, pc_w2, pc_b2, pc_w3, pc_b3, pc_ln_g, pc_ln_b,
           fu_w1, fu_b1, fu_w2, fu_b2, fu_ln_g, fu_ln_b):
    maskf = rel_dist_mask.astype(_F32)
    h = lambda a: a.astype(_BF16)
    img_h, pc_h = h(image_feature), h(point_cloud_feature)
    pimg_h, ppc_h = h(prev_image_feature), h(prev_point_cloud_feature)

    s_t = pl.pallas_call(
        _sim_body,
        grid=(_B,),
        in_specs=[
            pl.BlockSpec((1, _N, _IMG_D), lambda b: (b, 0, 0)),
            pl.BlockSpec((1, _N, _PC_D), lambda b: (b, 0, 0)),
            pl.BlockSpec((1, _N, _IMG_D), lambda b: (b, 0, 0)),
            pl.BlockSpec((1, _N, _PC_D), lambda b: (b, 0, 0)),
            pl.BlockSpec((1, _N, _N), lambda b: (b, 0, 0)),
        ],
        out_specs=pl.BlockSpec((1, _N, _N), lambda b: (b, 0, 0)),
        out_shape=jax.ShapeDtypeStruct((_B, _N, _N), _F32),
    )(img_h, pc_h, pimg_h, ppc_h, maskf)

    p_t, hit_flag = _sc_match(s_t)

    full = lambda a: pl.BlockSpec(a.shape, lambda b: (0,) * a.ndim)
    bat = lambda d: pl.BlockSpec((_BPG, _N, d), lambda b: (b, 0, 0))
    pspec = pl.BlockSpec((_BPG, _N, _N), lambda b: (b, 0, 0))
    bat2 = lambda d: pl.BlockSpec((_BPG2, _N, d), lambda b: (b, 0, 0))
    pspec2 = pl.BlockSpec((_BPG2, _N, _N), lambda b: (b, 0, 0))
    row = lambda a: a.reshape(1, -1)
    iw1a, iw1b = h(img_w1[:_IMG_D]), h(img_w1[_IMG_D:])
    pw1a, pw1b = h(pc_w1[:_PC_D]), h(pc_w1[_PC_D:])
    fw1a, fw1b = h(fu_w1[:_VIS_D]), h(fu_w1[_VIS_D:])
    iw2, iw3 = h(img_w2), h(img_w3)
    pw2, pw3 = h(pc_w2), h(pc_w3)
    fw2 = h(fu_w2)
    ib1, ib2, ib3 = row(img_b1), row(img_b2), row(img_b3)
    ilg, ilb = row(img_ln_g), row(img_ln_b)
    pb1, pb2, pb3 = row(pc_b1), row(pc_b2), row(pc_b3)
    plg, plb = row(pc_ln_g), row(pc_ln_b)
    fb1, fb2 = row(fu_b1), row(fu_b2)
    flg, flb = row(fu_ln_g), row(fu_ln_b)

    anyspec = pl.BlockSpec(memory_space=pl.ANY)
    hi = pl.pallas_call(
        _img_chain_body,
        grid=(_B // _BPG,),
        in_specs=[pl.BlockSpec(memory_space=pltpu.SMEM), pspec, bat(_IMG_D),
                  bat(_IMG_D), anyspec, anyspec, anyspec,
                  anyspec, full(ib1), full(ib2), full(ib3),
                  full(ilg), full(ilb)],
        out_specs=bat(_VIS_D),
        out_shape=jax.ShapeDtypeStruct((_B, _N, _VIS_D), _F32),
        scratch_shapes=[
            pltpu.VMEM((_IMG_D, _IMG_D), _BF16),
            pltpu.VMEM((_IMG_D, _IMG_D), _BF16),
            pltpu.VMEM((_IMG_D, _IMG_D), _BF16),
            pltpu.VMEM((_IMG_D, _VIS_D), _BF16),
            pltpu.SemaphoreType.DMA,
        ],
    )(hit_flag, p_t, img_h, pimg_h, iw1a, iw1b, iw2, iw3, ib1, ib2, ib3,
      ilg, ilb)

    vis, new_sp = pl.pallas_call(
        _pcfu_body,
        grid=(_B // _BPG2,),
        in_specs=[pl.BlockSpec(memory_space=pltpu.SMEM), pspec2,
                  bat2(_PC_D), bat2(_PC_D), bat2(_SP_D), bat2(_VIS_D),
                  anyspec, anyspec, anyspec, anyspec, anyspec, anyspec,
                  anyspec, full(pb1), full(pb2), full(pb3), full(plg),
                  full(plb), full(fb1), full(fb2), full(flg), full(flb)],
        out_specs=[bat2(_VIS_D), bat2(_SP_D)],
        out_shape=[
            jax.ShapeDtypeStruct((_B, _N, _VIS_D), _F32),
            jax.ShapeDtypeStruct((_B, _N, _SP_D), _F32),
        ],
        scratch_shapes=[
            pltpu.VMEM((_PC_D, _PC_D), _BF16),
            pltpu.VMEM((_PC_D, _PC_D), _BF16),
            pltpu.VMEM((_PC_D, _PC_D), _BF16),
            pltpu.VMEM((_PC_D, _VIS_D), _BF16),
            pltpu.VMEM((_VIS_D, _VIS_D), _BF16),
            pltpu.VMEM((_VIS_D, _VIS_D), _BF16),
            pltpu.VMEM((_VIS_D, _VIS_D), _BF16),
            pltpu.SemaphoreType.DMA,
        ],
    )(hit_flag, p_t, pc_h, ppc_h, prev_spatial, hi,
      pw1a, pw1b, pw2, pw3, fw1a, fw1b, fw2,
      pb1, pb2, pb3, plg, plb, fb1, fb2, flg, flb)

    return vis, new_sp


# speedup vs baseline: 1.0058x; 1.0058x over previous
import jax
import jax.numpy as jnp
from jax import lax
from jax.experimental import pallas as pl
from jax.experimental.pallas import tpu as pltpu
from jax.experimental.pallas import tpu_sc as plsc
import functools

_B, _N = 16, 256
_IMG_D, _PC_D = 2048, 768
_VIS_D, _SP_D = 768, 8
_THR = 0.5
_EPS = 1e-8
_NEG = -1e30
_F32 = jnp.float32
_BPG = 2
_BPG2 = 4
_BF16 = jnp.bfloat16


def _sim_body(img_ref, pc_ref, pimg_ref, ppc_ref, mask_ref, s_ref):
    img = img_ref[0]
    pc = pc_ref[0]
    pimg = pimg_ref[0]
    ppc = ppc_ref[0]
    dn = (((1,), (1,)), ((), ()))
    dot = lax.dot_general(img, pimg, dn, preferred_element_type=_F32)
    dot += lax.dot_general(pc, ppc, dn, preferred_element_type=_F32)
    sq = lambda a: jnp.sum(a.astype(_F32) ** 2, 1, keepdims=True)
    nf = jnp.maximum(jnp.sqrt(sq(img) + sq(pc)), _EPS)
    npr = jnp.maximum(jnp.sqrt(sq(pimg) + sq(ppc)), _EPS)
    sim = dot / (nf * npr.reshape(1, _N))
    s_ref[0] = jnp.where(mask_ref[0] != 0, sim, _NEG)


def _vperm16(x, perm):
    dnums = lax.GatherDimensionNumbers(offset_dims=(),
                                       collapsed_slice_dims=(0,),
                                       start_index_map=(0,))
    return lax.gather(x, perm.reshape(16, 1), dnums, slice_sizes=(1,),
                      mode=lax.GatherScatterMode.PROMISE_IN_BOUNDS)


def _bcast_max16(x):
    iota = lax.broadcasted_iota(jnp.int32, (16,), 0)
    for sh in (1, 2, 4, 8):
        x = jnp.maximum(x, _vperm16(x, jnp.bitwise_xor(iota, sh)))
    return x


def _bcast_min16i(x):
    iota = lax.broadcasted_iota(jnp.int32, (16,), 0)
    for sh in (1, 2, 4, 8):
        x = jnp.minimum(x, _vperm16(x, jnp.bitwise_xor(iota, sh)))
    return x


def _sc_match_body(s_hbm, p_hbm, flag_hbm, s_v, p_blk, vis_v, mx_v,
                   allmx_v, flag_v, mx_sh, sem):
    cid = lax.axis_index("c")
    sid = lax.axis_index("s")
    active = cid == 0
    iota = lax.broadcasted_iota(jnp.int32, (16,), 0)

    @pl.when(active)
    def _local_max():
        pltpu.sync_copy(s_hbm.at[sid], s_v)

        def rowmax(i, acc):
            for k in range(16):
                acc = jnp.maximum(acc, s_v[i, pl.ds(k * 16, 16)])
            return acc

        mx_v[...] = lax.fori_loop(0, _N, rowmax,
                                  jnp.full((16,), _NEG, _F32))
        pltpu.sync_copy(mx_v, mx_sh.at[sid])

    plsc.subcore_barrier()

    @pl.when(active)
    def _decide():
        pltpu.sync_copy(mx_sh, allmx_v)
        g = jnp.full((16,), _NEG, _F32)
        for r in range(16):
            g = jnp.maximum(g, allmx_v[r])
        any_hit = _bcast_max16(g)[0] >= _THR

        @pl.when(sid == 0)
        def _flag():
            flag_v[0] = jnp.ones((16,), jnp.int32) * any_hit.astype(jnp.int32)
            pltpu.sync_copy(flag_v, flag_hbm)

        @pl.when(any_hit)
        def _slow():
            for k in range(16):
                vis_v[pl.ds(k * 16, 16)] = jnp.zeros((16,), _F32)

            def blk(gblk, c0):
                def zrow(r, c1):
                    for k in range(16):
                        p_blk[r, pl.ds(k * 16, 16)] = jnp.zeros((16,), _F32)
                    return c1

                lax.fori_loop(0, 64, zrow, 0)

                def step(ii, c2):
                    i = gblk * 64 + ii
                    vmax = jnp.full((16,), _NEG, _F32)
                    for k in range(16):
                        vmax = jnp.maximum(
                            vmax, s_v[i, pl.ds(k * 16, 16)]
                            + vis_v[pl.ds(k * 16, 16)])
                    mv = _bcast_max16(vmax)
                    m = mv[0]
                    cmin = jnp.full((16,), 1024, jnp.int32)
                    for k in range(16):
                        v = (s_v[i, pl.ds(k * 16, 16)]
                             + vis_v[pl.ds(k * 16, 16)])
                        cmin = jnp.minimum(
                            cmin, jnp.where(v == mv, iota + k * 16, 1024))
                    jv = _bcast_min16i(cmin)

                    @pl.when(m >= _THR)
                    def _mark():
                        for k in range(16):
                            sel = (iota + k * 16) == jv
                            vis_v[pl.ds(k * 16, 16)] = jnp.where(
                                sel, _NEG, vis_v[pl.ds(k * 16, 16)])
                            p_blk[ii, pl.ds(k * 16, 16)] = jnp.where(
                                sel, 1.0, 0.0).astype(_F32)
                    return c2

                lax.fori_loop(0, 64, step, 0)
                pltpu.sync_copy(p_blk, p_hbm.at[sid, pl.ds(gblk * 64, 64)])
                return c0

            lax.fori_loop(0, 4, blk, 0)


def _sc_match(s_t):
    mesh = plsc.VectorSubcoreMesh(core_axis_name="c", subcore_axis_name="s")
    kern = functools.partial(
        pl.kernel, mesh=mesh,
        out_type=[jax.ShapeDtypeStruct((_B, _N, _N), _F32),
                  jax.ShapeDtypeStruct((1, 16), jnp.int32)],
        scratch_types=[
            pltpu.VMEM((_N, _N), _F32),
            pltpu.VMEM((64, _N), _F32),
            pltpu.VMEM((_N,), _F32),
            pltpu.VMEM((16,), _F32),
            pltpu.VMEM((16, 16), _F32),
            pltpu.VMEM((1, 16), jnp.int32),
            pltpu.VMEM_SHARED((16, 16), _F32),
            pltpu.SemaphoreType.DMA,
        ],
    )(_sc_match_body)
    return kern(s_t)


def _layer_norm(x, g, b):
    m = jnp.mean(x, axis=1, keepdims=True)
    v = jnp.mean((x - m) ** 2, axis=1, keepdims=True)
    return (x - m) / jnp.sqrt(v + 1e-5) * g + b


def _img_chain_body(flag_ref, p_ref, img_ref, pimg_ref, w1a_ref, w1b_ref,
                    w2_ref, w3_ref, b1_ref, b2_ref, b3_ref, g_ref, bb_ref,
                    hi_ref):
    x = img_ref[...].reshape(_BPG * _N, _IMG_D)
    base = jnp.dot(x, w1a_ref[...], preferred_element_type=_F32) + b1_ref[...]

    def _with_prev():
        oimg = jnp.concatenate(
            [jnp.dot(p_ref[k].astype(_BF16), pimg_ref[k],
                     preferred_element_type=_F32) for k in range(_BPG)], axis=0)
        return base + jnp.dot(oimg.astype(_BF16), w1b_ref[...],
                              preferred_element_type=_F32)

    h = lax.cond(flag_ref[0, 0] == 1, _with_prev, lambda: base)
    h = jnp.maximum(h, 0.0).astype(_BF16)
    h = jnp.dot(h, w2_ref[...], preferred_element_type=_F32) + b2_ref[...]
    h = jnp.dot(h.astype(_BF16), w3_ref[...], preferred_element_type=_F32)
    h = h + b3_ref[...]
    hi_ref[...] = _layer_norm(h, g_ref[...], bb_ref[...]).reshape(
        _BPG, _N, _VIS_D)


def _pcfu_body(flag_ref, p_ref, pc_ref, ppc_ref, psp_ref, hi_ref,
               pw1a_ref, pw1b_ref, pw2_ref, pw3_ref,
               fw1a_ref, fw1b_ref, fw2_ref,
               pb1_ref, pb2_ref, pb3_ref, plg_ref, plb_ref,
               fb1_ref, fb2_ref, flg_ref, flb_ref, vis_ref, sp_ref):
    matched = flag_ref[0, 0] == 1

    @pl.when(matched)
    def _sp_gather():
        for k in range(_BPG2):
            sp_ref[k] = jnp.dot(p_ref[k], psp_ref[k],
                                preferred_element_type=_F32)

    @pl.when(jnp.logical_not(matched))
    def _sp_zero():
        sp_ref[...] = jnp.zeros((_BPG2, _N, _SP_D), _F32)

    xpc = pc_ref[...].reshape(_BPG2 * _N, _PC_D)
    pbase = jnp.dot(xpc, pw1a_ref[...], preferred_element_type=_F32) + pb1_ref[...]

    def _with_prev():
        opc = jnp.concatenate(
            [jnp.dot(p_ref[k].astype(_BF16), ppc_ref[k],
                     preferred_element_type=_F32) for k in range(_BPG2)], axis=0)
        return pbase + jnp.dot(opc.astype(_BF16), pw1b_ref[...],
                               preferred_element_type=_F32)

    h = lax.cond(matched, _with_prev, lambda: pbase)
    h = jnp.maximum(h, 0.0).astype(_BF16)
    h = jnp.dot(h, pw2_ref[...], preferred_element_type=_F32) + pb2_ref[...]
    h = jnp.dot(h.astype(_BF16), pw3_ref[...], preferred_element_type=_F32)
    h = h + pb3_ref[...]
    hp = _layer_norm(h, plg_ref[...], plb_ref[...])

    h = (jnp.dot(hi_ref[...].reshape(_BPG2 * _N, _VIS_D).astype(_BF16),
                 fw1a_ref[...], preferred_element_type=_F32)
         + jnp.dot(hp.astype(_BF16), fw1b_ref[...], preferred_element_type=_F32)
         + fb1_ref[...])
    h = jnp.maximum(h, 0.0).astype(_BF16)
    h = jnp.dot(h, fw2_ref[...], preferred_element_type=_F32) + fb2_ref[...]
    vis_ref[...] = _layer_norm(h, flg_ref[...], flb_ref[...]).reshape(
        _BPG2, _N, _VIS_D)


def kernel(image_feature, point_cloud_feature, prev_image_feature,
           prev_point_cloud_feature, rel_dist_mask, prev_spatial,
           img_w1, img_b1, img_w2, img_b2, img_w3, img_b3, img_ln_g, img_ln_b,
           pc_w1, pc_b1, pc_w2, pc_b2, pc_w3, pc_b3, pc_ln_g, pc_ln_b,
           fu_w1, fu_b1, fu_w2, fu_b2, fu_ln_g, fu_ln_b):
    maskf = rel_dist_mask.astype(_F32)
    h = lambda a: a.astype(_BF16)
    img_h, pc_h = h(image_feature), h(point_cloud_feature)
    pimg_h, ppc_h = h(prev_image_feature), h(prev_point_cloud_feature)

    s_t = pl.pallas_call(
        _sim_body,
        grid=(_B,),
        in_specs=[
            pl.BlockSpec((1, _N, _IMG_D), lambda b: (b, 0, 0)),
            pl.BlockSpec((1, _N, _PC_D), lambda b: (b, 0, 0)),
            pl.BlockSpec((1, _N, _IMG_D), lambda b: (b, 0, 0)),
            pl.BlockSpec((1, _N, _PC_D), lambda b: (b, 0, 0)),
            pl.BlockSpec((1, _N, _N), lambda b: (b, 0, 0)),
        ],
        out_specs=pl.BlockSpec((1, _N, _N), lambda b: (b, 0, 0)),
        out_shape=jax.ShapeDtypeStruct((_B, _N, _N), _F32),
    )(img_h, pc_h, pimg_h, ppc_h, maskf)

    p_t, hit_flag = _sc_match(s_t)

    full = lambda a: pl.BlockSpec(a.shape, lambda b: (0,) * a.ndim)
    bat = lambda d: pl.BlockSpec((_BPG, _N, d), lambda b: (b, 0, 0))
    pspec = pl.BlockSpec((_BPG, _N, _N), lambda b: (b, 0, 0))
    bat2 = lambda d: pl.BlockSpec((_BPG2, _N, d), lambda b: (b, 0, 0))
    pspec2 = pl.BlockSpec((_BPG2, _N, _N), lambda b: (b, 0, 0))
    row = lambda a: a.reshape(1, -1)
    iw1a, iw1b = h(img_w1[:_IMG_D]), h(img_w1[_IMG_D:])
    pw1a, pw1b = h(pc_w1[:_PC_D]), h(pc_w1[_PC_D:])
    fw1a, fw1b = h(fu_w1[:_VIS_D]), h(fu_w1[_VIS_D:])
    iw2, iw3 = h(img_w2), h(img_w3)
    pw2, pw3 = h(pc_w2), h(pc_w3)
    fw2 = h(fu_w2)
    ib1, ib2, ib3 = row(img_b1), row(img_b2), row(img_b3)
    ilg, ilb = row(img_ln_g), row(img_ln_b)
    pb1, pb2, pb3 = row(pc_b1), row(pc_b2), row(pc_b3)
    plg, plb = row(pc_ln_g), row(pc_ln_b)
    fb1, fb2 = row(fu_b1), row(fu_b2)
    flg, flb = row(fu_ln_g), row(fu_ln_b)

    hi = pl.pallas_call(
        _img_chain_body,
        grid=(_B // _BPG,),
        in_specs=[pl.BlockSpec(memory_space=pltpu.SMEM), pspec, bat(_IMG_D),
                  bat(_IMG_D), full(iw1a), full(iw1b), full(iw2), full(iw3),
                  full(ib1), full(ib2), full(ib3), full(ilg), full(ilb)],
        out_specs=bat(_VIS_D),
        out_shape=jax.ShapeDtypeStruct((_B, _N, _VIS_D), _F32),
    )(hit_flag, p_t, img_h, pimg_h, iw1a, iw1b, iw2, iw3, ib1, ib2, ib3,
      ilg, ilb)

    vis, new_sp = pl.pallas_call(
        _pcfu_body,
        grid=(_B // _BPG2,),
        in_specs=[pl.BlockSpec(memory_space=pltpu.SMEM), pspec2,
                  bat2(_PC_D), bat2(_PC_D), bat2(_SP_D), bat2(_VIS_D),
                  full(pw1a), full(pw1b), full(pw2), full(pw3), full(fw1a),
                  full(fw1b), full(fw2), full(pb1), full(pb2), full(pb3),
                  full(plg), full(plb), full(fb1), full(fb2), full(flg),
                  full(flb)],
        out_specs=[bat2(_VIS_D), bat2(_SP_D)],
        out_shape=[
            jax.ShapeDtypeStruct((_B, _N, _VIS_D), _F32),
            jax.ShapeDtypeStruct((_B, _N, _SP_D), _F32),
        ],
    )(hit_flag, p_t, pc_h, ppc_h, prev_spatial, hi,
      pw1a, pw1b, pw2, pw3, fw1a, fw1b, fw2,
      pb1, pb2, pb3, plg, plb, fb1, fb2, flg, flb)

    return vis, new_sp
